# Initial kernel scaffold; baseline (speedup 1.0000x reference)
import jax
import jax.numpy as jnp
from jax.experimental import pallas as pl


def kernel(x, w1, b1, w2, b2, w3, b3, w_fc, b_fc):
    raise NotImplementedError("write your pallas kernel here")


# speedup vs baseline: 25.8677x; 25.8677x over previous
import functools

import jax
import jax.numpy as jnp
from jax.experimental import pallas as pl
from jax.experimental.pallas import tpu as pltpu


def _pack2x2(x):
    B, H, W, C = x.shape
    x = x.reshape(B, H // 2, 2, W // 2, 2, C)
    x = x.transpose(0, 1, 3, 2, 4, 5)
    return x.reshape(B, H // 2, W // 2, 4 * C)


def _eff_weights(w):
    C, cout = w.shape[2], w.shape[3]
    zero = jnp.zeros((C, cout), w.dtype)
    mats = []
    for ry in range(2):
        for rx in range(2):
            blocks = []
            for qy in range(2):
                for qx in range(2):
                    for py in range(2):
                        for px in range(2):
                            dy = 2 * qy + py + ry - 1
                            dx = 2 * qx + px + rx - 1
                            ok = 0 <= dy <= 2 and 0 <= dx <= 2
                            blocks.append(w[dy, dx] if ok else zero)
            mats.append(jnp.concatenate(blocks, axis=0))
    return jnp.stack(mats)


def _conv_pool_kernel(p_ref, w_ref, b_ref, o_ref, *, h2, w2):
    ib = p_ref.shape[0]
    c16 = w_ref.shape[1]
    cout = w_ref.shape[2]
    p = p_ref[...]
    p4 = jnp.concatenate(
        [p[:, :-1, :-1, :], p[:, :-1, 1:, :],
         p[:, 1:, :-1, :], p[:, 1:, 1:, :]], axis=-1)
    m = ib * h2 * w2
    pooled = None
    for ry in range(2):
        for rx in range(2):
            lhs = p4[:, ry:ry + h2, rx:rx + w2, :].reshape(m, c16)
            acc = jnp.dot(lhs, w_ref[ry * 2 + rx],
                          preferred_element_type=jnp.float32)
            pooled = acc if pooled is None else jnp.maximum(pooled, acc)
    y = pooled + b_ref[...]
    y = jnp.where(y > 0.0, y, jnp.exp(jnp.minimum(y, 0.0)) - 1.0)
    o_ref[...] = y.reshape(ib, h2 * w2, cout).astype(o_ref.dtype)


def _conv_stage(x, w, b, ib):
    B, H, W, C = x.shape
    h2, w2 = H // 2, W // 2
    cout = w.shape[-1]
    p = _pack2x2(x).astype(jnp.bfloat16)
    p = jnp.pad(p, ((0, 0), (1, 1), (1, 1), (0, 0)))
    wp = _eff_weights(w).astype(jnp.bfloat16)
    kern = functools.partial(_conv_pool_kernel, h2=h2, w2=w2)
    out = pl.pallas_call(
        kern,
        out_shape=jax.ShapeDtypeStruct((B, h2 * w2, cout), jnp.bfloat16),
        grid=(B // ib,),
        in_specs=[pl.BlockSpec((ib, h2 + 2, w2 + 2, 4 * C),
                               lambda i: (i, 0, 0, 0)),
                  pl.BlockSpec((4, 16 * C, cout), lambda i: (0, 0, 0)),
                  pl.BlockSpec((1, cout), lambda i: (0, 0))],
        out_specs=pl.BlockSpec((ib, h2 * w2, cout), lambda i: (i, 0, 0)),
        compiler_params=pltpu.CompilerParams(
            dimension_semantics=("parallel",),
            vmem_limit_bytes=64 * 1024 * 1024),
    )(p, wp, b.reshape(1, cout))
    return out.reshape(B, h2, w2, cout)


def _fc_kernel(x_ref, v_ref, b_ref, o_ref):
    acc = jax.lax.dot_general(
        x_ref[...], v_ref[...],
        dimension_numbers=(((1,), (1,)), ((), ())),
        preferred_element_type=jnp.float32)
    o_ref[...] = acc + b_ref[...]


def kernel(x, w1, b1, w2, b2, w3, b3, w_fc, b_fc):
    xh = jnp.transpose(x, (0, 2, 3, 1))
    a = _conv_stage(xh, w1, b1, 2)
    a = _conv_stage(a, w2, b2, 8)
    a = _conv_stage(a, w3, b3, 8)
    B = a.shape[0]
    feat = a.reshape(B, 16 * 16 * 128)
    n_cls, k_fc = w_fc.shape
    v = w_fc.reshape(n_cls, 128, 256).transpose(0, 2, 1).reshape(n_cls, k_fc)
    logits = pl.pallas_call(
        _fc_kernel,
        out_shape=jax.ShapeDtypeStruct((B, n_cls), jnp.float32),
        grid=(1,),
        in_specs=[pl.BlockSpec((B, k_fc), lambda i: (0, 0)),
                  pl.BlockSpec((n_cls, k_fc), lambda i: (0, 0)),
                  pl.BlockSpec((1, n_cls), lambda i: (0, 0))],
        out_specs=pl.BlockSpec((B, n_cls), lambda i: (0, 0)),
        compiler_params=pltpu.CompilerParams(
            dimension_semantics=("arbitrary",)),
    )(feat, v.astype(jnp.bfloat16), b_fc.reshape(1, n_cls))
    return logits


# speedup vs baseline: 26.0147x; 1.0057x over previous
import functools

import jax
import jax.numpy as jnp
from jax.experimental import pallas as pl
from jax.experimental.pallas import tpu as pltpu


def _pack2x2(x):
    B, H, W, C = x.shape
    x = x.reshape(B, H // 2, 2, W // 2, 2, C)
    x = x.transpose(0, 1, 3, 2, 4, 5)
    return x.reshape(B, H // 2, W // 2, 4 * C)


def _eff_weights(w):
    C, cout = w.shape[2], w.shape[3]
    zero = jnp.zeros((C, cout), w.dtype)
    mats = []
    for ry in range(2):
        for rx in range(2):
            blocks = []
            for qy in range(2):
                for qx in range(2):
                    for py in range(2):
                        for px in range(2):
                            dy = 2 * qy + py + ry - 1
                            dx = 2 * qx + px + rx - 1
                            ok = 0 <= dy <= 2 and 0 <= dx <= 2
                            blocks.append(w[dy, dx] if ok else zero)
            mats.append(jnp.concatenate(blocks, axis=0))
    return jnp.stack(mats)


def _conv_pool_kernel(p_ref, w_ref, b_ref, o_ref, *, h2, w2):
    ib = p_ref.shape[0]
    c16 = w_ref.shape[1]
    cout = w_ref.shape[2]
    p = p_ref[...]
    p4 = jnp.concatenate(
        [p[:, :-1, :-1, :], p[:, :-1, 1:, :],
         p[:, 1:, :-1, :], p[:, 1:, 1:, :]], axis=-1)
    m = ib * h2 * w2
    pooled = None
    for ry in range(2):
        for rx in range(2):
            lhs = p4[:, ry:ry + h2, rx:rx + w2, :].reshape(m, c16)
            acc = jnp.dot(lhs, w_ref[ry * 2 + rx],
                          preferred_element_type=jnp.float32)
            pooled = acc if pooled is None else jnp.maximum(pooled, acc)
    y = pooled + b_ref[...]
    y = jnp.where(y > 0.0, y, jnp.exp(jnp.minimum(y, 0.0)) - 1.0)
    o_ref[...] = y.reshape(ib, h2 * w2, cout).astype(o_ref.dtype)


def _conv_stage(x, w, b, ib):
    B, H, W, C = x.shape
    h2, w2 = H // 2, W // 2
    cout = w.shape[-1]
    p = _pack2x2(x).astype(jnp.bfloat16)
    p = jnp.pad(p, ((0, 0), (1, 1), (1, 1), (0, 0)))
    wp = _eff_weights(w).astype(jnp.bfloat16)
    kern = functools.partial(_conv_pool_kernel, h2=h2, w2=w2)
    out = pl.pallas_call(
        kern,
        out_shape=jax.ShapeDtypeStruct((B, h2 * w2, cout), jnp.bfloat16),
        grid=(B // ib,),
        in_specs=[pl.BlockSpec((ib, h2 + 2, w2 + 2, 4 * C),
                               lambda i: (i, 0, 0, 0)),
                  pl.BlockSpec((4, 16 * C, cout), lambda i: (0, 0, 0)),
                  pl.BlockSpec((1, cout), lambda i: (0, 0))],
        out_specs=pl.BlockSpec((ib, h2 * w2, cout), lambda i: (i, 0, 0)),
        compiler_params=pltpu.CompilerParams(
            dimension_semantics=("parallel",),
            vmem_limit_bytes=64 * 1024 * 1024),
    )(p, wp, b.reshape(1, cout))
    return out.reshape(B, h2, w2, cout)


def _fc_kernel(x_ref, v_ref, b_ref, o_ref):
    acc = jax.lax.dot_general(
        x_ref[...], v_ref[...],
        dimension_numbers=(((1,), (1,)), ((), ())),
        preferred_element_type=jnp.float32)
    o_ref[...] = acc + b_ref[...]


def kernel(x, w1, b1, w2, b2, w3, b3, w_fc, b_fc):
    B, c_in, H, _ = x.shape
    c4 = w3.shape[-1]
    h3 = H // 8
    ib1 = 4 if B % 4 == 0 else 1
    ib2 = 8 if B % 8 == 0 else 1
    xh = jnp.transpose(x, (0, 2, 3, 1))
    a = _conv_stage(xh, w1, b1, ib1)
    a = _conv_stage(a, w2, b2, ib2)
    a = _conv_stage(a, w3, b3, ib2)
    feat = a.reshape(B, h3 * h3 * c4)
    n_cls, k_fc = w_fc.shape
    v = w_fc.reshape(n_cls, c4, h3 * h3).transpose(0, 2, 1).reshape(n_cls, k_fc)
    logits = pl.pallas_call(
        _fc_kernel,
        out_shape=jax.ShapeDtypeStruct((B, n_cls), jnp.float32),
        grid=(1,),
        in_specs=[pl.BlockSpec((B, k_fc), lambda i: (0, 0)),
                  pl.BlockSpec((n_cls, k_fc), lambda i: (0, 0)),
                  pl.BlockSpec((1, n_cls), lambda i: (0, 0))],
        out_specs=pl.BlockSpec((B, n_cls), lambda i: (0, 0)),
        compiler_params=pltpu.CompilerParams(
            dimension_semantics=("arbitrary",)),
    )(feat, v.astype(jnp.bfloat16), b_fc.reshape(1, n_cls))
    return logits
